# baseline (speedup 1.0000x reference)
import functools

import jax
import jax.numpy as jnp
from jax import lax
from jax.experimental import pallas as pl
from jax.experimental.pallas import tpu as pltpu
from jax.experimental.pallas import tpu_sc as plsc

VOCAB = 1000000
EMBED_DIM = 64
BATCH = 4096
HIST_LEN = 200

NC = 2
NS = 16
NW = NC * NS

B_TOTAL = BATCH * HIST_LEN
B_PER_W = B_TOTAL // NW
CHUNK = 512
N_CHUNKS = B_PER_W // CHUNK


def _gather_body(idx_hbm, table_hbm, out_hbm, idx_v, rows_v, sem):
    wid = lax.axis_index("s") * NC + lax.axis_index("c")
    base = wid * B_PER_W

    def step(j, _):
        off = base + j * CHUNK
        pltpu.sync_copy(idx_hbm.at[pl.ds(off, CHUNK)], idx_v)
        pltpu.async_copy(table_hbm.at[idx_v], rows_v, sem).wait()
        pltpu.sync_copy(rows_v, out_hbm.at[pl.ds(off, CHUNK)])
        return ()

    lax.fori_loop(0, N_CHUNKS, step, (), unroll=False)


@functools.partial(jax.jit, static_argnames=())
def _gather(idx_flat, table):
    mesh = plsc.VectorSubcoreMesh(core_axis_name="c", subcore_axis_name="s")
    kern = pl.kernel(
        _gather_body,
        out_type=jax.ShapeDtypeStruct((B_TOTAL, EMBED_DIM), jnp.float32),
        mesh=mesh,
        scratch_types=[
            pltpu.VMEM((CHUNK,), jnp.int32),
            pltpu.VMEM((CHUNK, EMBED_DIM), jnp.float32),
            pltpu.SemaphoreType.DMA,
        ],
        compiler_params=pltpu.CompilerParams(use_tc_tiling_on_sc=False),
    )
    return kern(idx_flat, table)


def kernel(inputs, embeddings):
    idx_flat = inputs.reshape(-1).astype(jnp.int32)
    out = _gather(idx_flat, embeddings)
    return out.reshape(inputs.shape + (EMBED_DIM,))


# speedup vs baseline: 1.0462x; 1.0462x over previous
import functools

import jax
import jax.numpy as jnp
from jax import lax
from jax.experimental import pallas as pl
from jax.experimental.pallas import tpu as pltpu
from jax.experimental.pallas import tpu_sc as plsc

VOCAB = 1000000
EMBED_DIM = 64
BATCH = 4096
HIST_LEN = 200

NC = 2
NS = 16
NW = NC * NS

B_TOTAL = BATCH * HIST_LEN
B_PER_W = B_TOTAL // NW
CHUNK = 512
N_CHUNKS = B_PER_W // CHUNK
NBUF = 2


def _gather_body(idx_hbm, table_hbm, out_hbm, idx_v, rows_v, gsems, ssems):
    wid = lax.axis_index("s") * NC + lax.axis_index("c")
    base = wid * B_PER_W

    pltpu.sync_copy(idx_hbm.at[pl.ds(base, B_PER_W)], idx_v)

    def start_gather(j, b):
        pltpu.async_copy(
            table_hbm.at[idx_v.at[pl.ds(j * CHUNK, CHUNK)]],
            rows_v.at[b],
            gsems.at[b],
        )

    def start_store(j, b):
        pltpu.async_copy(
            rows_v.at[b],
            out_hbm.at[pl.ds(base + j * CHUNK, CHUNK)],
            ssems.at[b],
        )

    def wait_gather(j, b):
        pltpu.make_async_copy(
            table_hbm.at[idx_v.at[pl.ds(j * CHUNK, CHUNK)]],
            rows_v.at[b],
            gsems.at[b],
        ).wait()

    def wait_store(j, b):
        pltpu.make_async_copy(
            rows_v.at[b],
            out_hbm.at[pl.ds(base + j * CHUNK, CHUNK)],
            ssems.at[b],
        ).wait()

    start_gather(0, 0)

    def step(j, _):
        nb = lax.rem(j + 1, NBUF)
        b = lax.rem(j, NBUF)

        @pl.when(j + 1 < N_CHUNKS)
        def _():
            @pl.when(j + 1 >= NBUF)
            def _():
                wait_store(j + 1 - NBUF, nb)
            start_gather(j + 1, nb)

        wait_gather(j, b)
        start_store(j, b)
        return ()

    lax.fori_loop(0, N_CHUNKS, step, (), unroll=False)

    wait_store(N_CHUNKS - NBUF, lax.rem(N_CHUNKS - NBUF, NBUF))
    wait_store(N_CHUNKS - 1, lax.rem(N_CHUNKS - 1, NBUF))


@jax.jit
def _gather(idx_flat, table):
    mesh = plsc.VectorSubcoreMesh(core_axis_name="c", subcore_axis_name="s")
    kern = pl.kernel(
        _gather_body,
        out_type=jax.ShapeDtypeStruct((B_TOTAL, EMBED_DIM), jnp.float32),
        mesh=mesh,
        scratch_types=[
            pltpu.VMEM((B_PER_W,), jnp.int32),
            pltpu.VMEM((NBUF, CHUNK, EMBED_DIM), jnp.float32),
            pltpu.SemaphoreType.DMA((NBUF,)),
            pltpu.SemaphoreType.DMA((NBUF,)),
        ],
        compiler_params=pltpu.CompilerParams(use_tc_tiling_on_sc=False),
    )
    return kern(idx_flat, table)


def kernel(inputs, embeddings):
    idx_flat = inputs.reshape(-1).astype(jnp.int32)
    out = _gather(idx_flat, embeddings)
    return out.reshape(inputs.shape + (EMBED_DIM,))
